# baseline (speedup 1.0000x reference)
import dataclasses
import functools

import jax
import jax.numpy as jnp
from jax import lax
from jax.experimental import pallas as pl
from jax.experimental.pallas import tpu as pltpu
from jax.experimental.pallas import tpu_sc as plsc

E_CNT = 100000
R_CNT = 1000
DIM = 128
B = 16384

NC = 2
NS = 16
L = 16
NW = NC * NS
PER_W = B // NW
C = 16
NCHUNK = PER_W // C


def _sin_poly(x):
    x2 = x * x
    p = x2 * (1.0 / 362880.0) - (1.0 / 5040.0)
    p = x2 * p + (1.0 / 120.0)
    p = x2 * p - (1.0 / 6.0)
    return x + x * (x2 * p)


def _sc_scores(s, o, r, t, e_embed, r_embed, d_frq, h_frq, d_phi, h_phi,
               d_amp, h_amp):
    mesh = plsc.VectorSubcoreMesh(core_axis_name="c", subcore_axis_name="s")

    cp = pltpu.CompilerParams()
    if "needs_layout_passes" in pltpu.CompilerParams.__dataclass_fields__:
        cp = dataclasses.replace(cp, needs_layout_passes=False)

    row_buf = pltpu.VMEM((C, DIM), jnp.float32)

    @functools.partial(
        pl.kernel,
        out_type=jax.ShapeDtypeStruct((B,), jnp.float32),
        mesh=mesh,
        compiler_params=cp,
        scratch_types=[
            pltpu.VMEM((PER_W,), jnp.int32),
            pltpu.VMEM((PER_W,), jnp.int32),
            pltpu.VMEM((PER_W,), jnp.int32),
            pltpu.VMEM((PER_W, 2), jnp.float32),
            row_buf, row_buf,
            pltpu.VMEM((C, 2 * DIM), jnp.float32),
            row_buf, row_buf, row_buf,
            row_buf, row_buf, row_buf,
            row_buf, row_buf, row_buf,
            row_buf, row_buf, row_buf,
            pltpu.VMEM((C,), jnp.float32),
            pltpu.SemaphoreType.DMA,
        ],
    )
    def k(s_hbm, o_hbm, r_hbm, t_hbm, e_hbm, re_hbm, dfrq_hbm, hfrq_hbm,
          dphi_hbm, hphi_hbm, damp_hbm, hamp_hbm, out_hbm,
          s_v, o_v, r_v, t_v, se_b, oe_b, re_b,
          sdf_b, sdp_b, sda_b, shf_b, shp_b, sha_b,
          odf_b, odp_b, oda_b, ohf_b, ohp_b, oha_b,
          out_v, sem):
        wid = lax.axis_index("s") * NC + lax.axis_index("c")
        base = wid * PER_W
        pltpu.sync_copy(s_hbm.at[pl.ds(base, PER_W)], s_v)
        pltpu.sync_copy(o_hbm.at[pl.ds(base, PER_W)], o_v)
        pltpu.sync_copy(r_hbm.at[pl.ds(base, PER_W)], r_v)
        pltpu.sync_copy(t_hbm.at[pl.ds(base, PER_W)], t_v)

        eidx = lax.iota(jnp.int32, L)

        @pl.loop(0, NCHUNK)
        def _chunk(g):
            cb = pl.multiple_of(g * C, C)
            s_idx = s_v.at[pl.ds(cb, C)]
            o_idx = o_v.at[pl.ds(cb, C)]
            r_idx = r_v.at[pl.ds(cb, C)]
            cps = [
                pltpu.async_copy(e_hbm.at[s_idx], se_b, sem),
                pltpu.async_copy(e_hbm.at[o_idx], oe_b, sem),
                pltpu.async_copy(re_hbm.at[r_idx], re_b, sem),
                pltpu.async_copy(dfrq_hbm.at[s_idx], sdf_b, sem),
                pltpu.async_copy(dphi_hbm.at[s_idx], sdp_b, sem),
                pltpu.async_copy(damp_hbm.at[s_idx], sda_b, sem),
                pltpu.async_copy(hfrq_hbm.at[s_idx], shf_b, sem),
                pltpu.async_copy(hphi_hbm.at[s_idx], shp_b, sem),
                pltpu.async_copy(hamp_hbm.at[s_idx], sha_b, sem),
                pltpu.async_copy(dfrq_hbm.at[o_idx], odf_b, sem),
                pltpu.async_copy(dphi_hbm.at[o_idx], odp_b, sem),
                pltpu.async_copy(damp_hbm.at[o_idx], oda_b, sem),
                pltpu.async_copy(hfrq_hbm.at[o_idx], ohf_b, sem),
                pltpu.async_copy(hphi_hbm.at[o_idx], ohp_b, sem),
                pltpu.async_copy(hamp_hbm.at[o_idx], oha_b, sem),
            ]
            for cp in cps:
                cp.wait()

            d_vec = plsc.load_gather(t_v, [cb + eidx, jnp.zeros((L,), jnp.int32)])
            h_vec = plsc.load_gather(t_v, [cb + eidx, jnp.ones((L,), jnp.int32)])

            def body(j, acc):
                jj = jnp.full((L,), j, jnp.int32)
                se = plsc.load_gather(se_b, [eidx, jj])
                oe = plsc.load_gather(oe_b, [eidx, jj])
                r1 = plsc.load_gather(re_b, [eidx, jj])
                r2 = plsc.load_gather(re_b, [eidx, jj + DIM])
                t_s = (plsc.load_gather(sda_b, [eidx, jj])
                       * _sin_poly(plsc.load_gather(sdf_b, [eidx, jj]) * d_vec
                                   + plsc.load_gather(sdp_b, [eidx, jj]))
                       + plsc.load_gather(sha_b, [eidx, jj])
                       * _sin_poly(plsc.load_gather(shf_b, [eidx, jj]) * h_vec
                                   + plsc.load_gather(shp_b, [eidx, jj])))
                t_o = (plsc.load_gather(oda_b, [eidx, jj])
                       * _sin_poly(plsc.load_gather(odf_b, [eidx, jj]) * d_vec
                                   + plsc.load_gather(odp_b, [eidx, jj]))
                       + plsc.load_gather(oha_b, [eidx, jj])
                       * _sin_poly(plsc.load_gather(ohf_b, [eidx, jj]) * h_vec
                                   + plsc.load_gather(ohp_b, [eidx, jj])))
                return (acc + jnp.abs(se + r1 - oe)
                        + jnp.abs(t_s + r2 - t_o))

            acc = lax.fori_loop(0, DIM, body, jnp.zeros((L,), jnp.float32))
            out_v[...] = -acc
            pltpu.sync_copy(out_v, out_hbm.at[pl.ds(base + cb, C)])

    return k(s, o, r, t, e_embed, r_embed, d_frq, h_frq, d_phi, h_phi,
             d_amp, h_amp)


def kernel(s, o, r, t, e_embed, r_embed, d_frq_embed, h_frq_embed,
           d_phi_embed, h_phi_embed, d_amp_embed, h_amp_embed):
    s = s.astype(jnp.int32)
    o = o.astype(jnp.int32)
    r = r.astype(jnp.int32)
    return _sc_scores(s, o, r, t, e_embed, r_embed, d_frq_embed,
                      h_frq_embed, d_phi_embed, h_phi_embed,
                      d_amp_embed, h_amp_embed)
